# Initial kernel scaffold; baseline (speedup 1.0000x reference)
import jax
import jax.numpy as jnp
from jax.experimental import pallas as pl


def kernel(p, x, W, gamma, beta):
    raise NotImplementedError("write your pallas kernel here")


# speedup vs baseline: 1.5463x; 1.5463x over previous
import functools

import jax
import jax.numpy as jnp
from jax.experimental import pallas as pl
from jax.experimental.pallas import tpu as pltpu

_B, _C, _N = 8, 64, 100000
_TN = 25600
_NB = -(-_N // _TN)
_EPS = 1e-5


def _stats_kernel(x_ref, w_ref, g_ref, b_ref, scale_ref, shift_ref,
                  c_acc, s_acc):
    bi = pl.program_id(0)
    ni = pl.program_id(1)

    @pl.when((bi == 0) & (ni == 0))
    def _init():
        c_acc[...] = jnp.zeros_like(c_acc)
        s_acc[...] = jnp.zeros_like(s_acc)

    col = jax.lax.broadcasted_iota(jnp.int32, (_C, _TN), 1)
    valid = col < (_N - ni * _TN)
    xb = jnp.where(valid, x_ref[0], 0.0)
    c_acc[...] += jax.lax.dot_general(
        xb, xb, (((1,), (1,)), ((), ())),
        preferred_element_type=jnp.float32)
    s_acc[...] += jnp.sum(xb, axis=1, keepdims=True)

    @pl.when((bi == _B - 1) & (ni == _NB - 1))
    def _finish():
        cnt = jnp.float32(_B * _N)
        w = w_ref[...]
        mean = jnp.dot(w, s_acc[...], preferred_element_type=jnp.float32) / cnt
        a = jnp.dot(w, c_acc[...], preferred_element_type=jnp.float32)
        esq = jnp.sum(a * w, axis=1, keepdims=True) / cnt
        var = esq - mean * mean
        inv = g_ref[...] * jax.lax.rsqrt(var + _EPS)
        scale_ref[...] = inv
        shift_ref[...] = b_ref[...] - mean * inv


def _apply_kernel(x_ref, w_ref, scale_ref, shift_ref, o_ref):
    y = jnp.dot(w_ref[...], x_ref[0], preferred_element_type=jnp.float32)
    o_ref[0] = jnp.maximum(y * scale_ref[...] + shift_ref[...], 0.0)


@functools.partial(jax.jit, static_argnames=())
def _run(x, W, gamma, beta):
    g2 = gamma.reshape(_C, 1)
    b2 = beta.reshape(_C, 1)

    scale, shift = pl.pallas_call(
        _stats_kernel,
        grid=(_B, _NB),
        in_specs=[
            pl.BlockSpec((1, _C, _TN), lambda b, n: (b, 0, n)),
            pl.BlockSpec((_C, _C), lambda b, n: (0, 0)),
            pl.BlockSpec((_C, 1), lambda b, n: (0, 0)),
            pl.BlockSpec((_C, 1), lambda b, n: (0, 0)),
        ],
        out_specs=[
            pl.BlockSpec((_C, 1), lambda b, n: (0, 0)),
            pl.BlockSpec((_C, 1), lambda b, n: (0, 0)),
        ],
        out_shape=[
            jax.ShapeDtypeStruct((_C, 1), jnp.float32),
            jax.ShapeDtypeStruct((_C, 1), jnp.float32),
        ],
        scratch_shapes=[
            pltpu.VMEM((_C, _C), jnp.float32),
            pltpu.VMEM((_C, 1), jnp.float32),
        ],
    )(x, W, g2, b2)

    y = pl.pallas_call(
        _apply_kernel,
        grid=(_B, _NB),
        in_specs=[
            pl.BlockSpec((1, _C, _TN), lambda b, n: (b, 0, n)),
            pl.BlockSpec((_C, _C), lambda b, n: (0, 0)),
            pl.BlockSpec((_C, 1), lambda b, n: (0, 0)),
            pl.BlockSpec((_C, 1), lambda b, n: (0, 0)),
        ],
        out_specs=pl.BlockSpec((1, _C, _TN), lambda b, n: (b, 0, n)),
        out_shape=jax.ShapeDtypeStruct((_B, _C, _N), jnp.float32),
    )(x, W, scale, shift)

    return y


def kernel(p, x, W, gamma, beta):
    return (p, _run(x, W, gamma, beta))
